# baseline (device time: 10632 ns/iter reference)
import jax
import jax.numpy as jnp
from jax import lax
from jax.experimental import pallas as pl
from jax.experimental.pallas import tpu as pltpu

K = 8
NEG = float("-inf")


def _topk_desc_distinct(x, k):
    vals = []
    cur = x
    for _ in range(k):
        m = jnp.max(cur, axis=1, keepdims=True)
        vals.append(m)
        cur = jnp.where(cur == m, NEG, cur)
    return jnp.concatenate(vals, axis=1)


def _distinctify(xv):
    rows, n = xv.shape
    col = lax.broadcasted_iota(jnp.int32, (rows, n), 1)
    bits = lax.bitcast_convert_type(xv, jnp.int32)
    bits = jnp.bitwise_or(jnp.bitwise_and(bits, jnp.int32(-1024)), col)
    return lax.bitcast_convert_type(bits, jnp.float32)


def _merge_sorted8(a, b):
    b_rev = jnp.concatenate(
        [b[:, i:i + 1] for i in range(K - 1, -1, -1)], axis=1
    )
    L = jnp.maximum(a, b_rev)
    hi = jnp.maximum(L[:, :4], L[:, 4:])
    lo = jnp.minimum(L[:, :4], L[:, 4:])
    L = jnp.concatenate([hi, lo], axis=1)
    parts = []
    for s in (0, 4):
        x0 = L[:, s:s + 2]
        x1 = L[:, s + 2:s + 4]
        parts += [jnp.maximum(x0, x1), jnp.minimum(x0, x1)]
    L = jnp.concatenate(parts, axis=1)
    parts = []
    for s in (0, 2, 4, 6):
        x0 = L[:, s:s + 1]
        x1 = L[:, s + 1:s + 2]
        parts += [jnp.maximum(x0, x1), jnp.minimum(x0, x1)]
    return jnp.concatenate(parts, axis=1)


def kernel(x):
    rows, n = x.shape
    half = n // 2

    def body(x_hbm, out_ref, xv_ref, comm_ref, dma_sems, send_sems, recv_sems):
        my_x = lax.axis_index("x")
        my_y = lax.axis_index("y")
        nbr = (1 - my_x, my_y)

        cp0 = pltpu.make_async_copy(
            x_hbm.at[:, pl.ds(0, half)], xv_ref.at[0], dma_sems.at[0]
        )
        cp1 = pltpu.make_async_copy(
            x_hbm.at[:, pl.ds(half, half)], xv_ref.at[1], dma_sems.at[1]
        )
        cp0.start()
        cp1.start()

        barrier_sem = pltpu.get_barrier_semaphore()
        pl.semaphore_signal(
            barrier_sem, inc=1, device_id=nbr,
            device_id_type=pl.DeviceIdType.MESH,
        )

        def exchange(slot):
            return pltpu.make_async_remote_copy(
                src_ref=comm_ref.at[slot],
                dst_ref=comm_ref.at[2 + slot],
                send_sem=send_sems.at[slot],
                recv_sem=recv_sems.at[slot],
                device_id=nbr,
                device_id_type=pl.DeviceIdType.MESH,
            )

        cp0.wait()
        h0 = _topk_desc_distinct(_distinctify(xv_ref[0]), K)
        comm_ref[0, :, :] = h0
        pl.semaphore_wait(barrier_sem, 1)
        rdma0 = exchange(0)
        rdma0.start()

        cp1.wait()
        h1 = _topk_desc_distinct(_distinctify(xv_ref[1]), K)
        comm_ref[1, :, :] = h1
        rdma1 = exchange(1)
        rdma1.start()

        own = _merge_sorted8(h0, h1)

        rdma0.wait()
        rdma1.wait()
        rem = _merge_sorted8(comm_ref[2, :, :], comm_ref[3, :, :])
        out_ref[:, :] = _merge_sorted8(own, rem)

    return pl.pallas_call(
        body,
        out_shape=jax.ShapeDtypeStruct((rows, K), jnp.float32),
        in_specs=[pl.BlockSpec(memory_space=pl.ANY)],
        out_specs=pl.BlockSpec(memory_space=pltpu.VMEM),
        scratch_shapes=[
            pltpu.VMEM((2, rows, half), jnp.float32),
            pltpu.VMEM((4, rows, K), jnp.float32),
            pltpu.SemaphoreType.DMA((2,)),
            pltpu.SemaphoreType.DMA((2,)),
            pltpu.SemaphoreType.DMA((2,)),
        ],
        compiler_params=pltpu.CompilerParams(collective_id=0),
    )(x)


# device time: 9234 ns/iter; 1.1514x vs baseline; 1.1514x over previous
import jax
import jax.numpy as jnp
from jax import lax
from jax.experimental import pallas as pl
from jax.experimental.pallas import tpu as pltpu

K = 8
NEG = float("-inf")


def _distinctify(xv):
    rows, n = xv.shape
    col = lax.broadcasted_iota(jnp.int32, (rows, n), 1)
    bits = lax.bitcast_convert_type(xv, jnp.int32)
    bits = jnp.bitwise_or(jnp.bitwise_and(bits, jnp.int32(-1024)), col)
    return lax.bitcast_convert_type(bits, jnp.float32)


def _topk_desc_distinct(x, k):
    vals = []
    cur = x
    for _ in range(k):
        m = jnp.max(cur, axis=1, keepdims=True)
        vals.append(m)
        cur = jnp.where(cur == m, NEG, cur)
    return jnp.concatenate(vals, axis=1)


def _merge_sorted8(a, b):
    b_rev = jnp.concatenate(
        [b[:, i:i + 1] for i in range(K - 1, -1, -1)], axis=1
    )
    L = jnp.maximum(a, b_rev)
    hi = jnp.maximum(L[:, :4], L[:, 4:])
    lo = jnp.minimum(L[:, :4], L[:, 4:])
    L = jnp.concatenate([hi, lo], axis=1)
    parts = []
    for s in (0, 4):
        x0 = L[:, s:s + 2]
        x1 = L[:, s + 2:s + 4]
        parts += [jnp.maximum(x0, x1), jnp.minimum(x0, x1)]
    L = jnp.concatenate(parts, axis=1)
    parts = []
    for s in (0, 2, 4, 6):
        x0 = L[:, s:s + 1]
        x1 = L[:, s + 1:s + 2]
        parts += [jnp.maximum(x0, x1), jnp.minimum(x0, x1)]
    return jnp.concatenate(parts, axis=1)


def kernel(x):
    rows, n = x.shape
    hrows = rows // 2

    def body(x_hbm, out_ref, xv_ref, comm_ref, dma_sems, send_sem, recv_sem):
        my_x = lax.axis_index("x")
        my_y = lax.axis_index("y")
        nbr = (1 - my_x, my_y)

        cp0 = pltpu.make_async_copy(
            x_hbm.at[pl.ds(0, hrows), :],
            xv_ref.at[pl.ds(0, hrows), :],
            dma_sems.at[0],
        )
        cp1 = pltpu.make_async_copy(
            x_hbm.at[pl.ds(hrows, hrows), :],
            xv_ref.at[pl.ds(hrows, hrows), :],
            dma_sems.at[1],
        )
        cp0.start()
        cp1.start()

        barrier_sem = pltpu.get_barrier_semaphore()
        pl.semaphore_signal(
            barrier_sem, inc=1, device_id=nbr,
            device_id_type=pl.DeviceIdType.MESH,
        )

        cp0.wait()
        top = _topk_desc_distinct(_distinctify(xv_ref[:hrows, :]), K)
        cp1.wait()
        bot = _topk_desc_distinct(_distinctify(xv_ref[hrows:, :]), K)
        local = jnp.concatenate([top, bot], axis=0)
        comm_ref[0, :, :] = local

        pl.semaphore_wait(barrier_sem, 1)
        rdma = pltpu.make_async_remote_copy(
            src_ref=comm_ref.at[0],
            dst_ref=comm_ref.at[1],
            send_sem=send_sem,
            recv_sem=recv_sem,
            device_id=nbr,
            device_id_type=pl.DeviceIdType.MESH,
        )
        rdma.start()
        rdma.wait_recv()
        out_ref[:, :] = _merge_sorted8(local, comm_ref[1, :, :])
        rdma.wait_send()

    return pl.pallas_call(
        body,
        out_shape=jax.ShapeDtypeStruct((rows, K), jnp.float32),
        in_specs=[pl.BlockSpec(memory_space=pl.ANY)],
        out_specs=pl.BlockSpec(memory_space=pltpu.VMEM),
        scratch_shapes=[
            pltpu.VMEM((rows, n), jnp.float32),
            pltpu.VMEM((2, rows, K), jnp.float32),
            pltpu.SemaphoreType.DMA((2,)),
            pltpu.SemaphoreType.DMA,
            pltpu.SemaphoreType.DMA,
        ],
        compiler_params=pltpu.CompilerParams(collective_id=0),
    )(x)


# device time: 8585 ns/iter; 1.2384x vs baseline; 1.0756x over previous
import jax
import jax.numpy as jnp
from jax import lax
from jax.experimental import pallas as pl
from jax.experimental.pallas import tpu as pltpu

K = 8
NEG = float("-inf")


def _distinctify(xv):
    rows, n = xv.shape
    col = lax.broadcasted_iota(jnp.int32, (rows, n), 1)
    bits = lax.bitcast_convert_type(xv, jnp.int32)
    bits = jnp.bitwise_or(jnp.bitwise_and(bits, jnp.int32(-1024)), col)
    return lax.bitcast_convert_type(bits, jnp.float32)


def _topk_desc_distinct(x, k):
    vals = []
    cur = x
    for _ in range(k):
        m = jnp.max(cur, axis=1, keepdims=True)
        vals.append(m)
        cur = jnp.where(cur == m, NEG, cur)
    return jnp.concatenate(vals, axis=1)


def _merge_sorted8(a, b):
    b_rev = jnp.concatenate(
        [b[:, i:i + 1] for i in range(K - 1, -1, -1)], axis=1
    )
    L = jnp.maximum(a, b_rev)
    hi = jnp.maximum(L[:, :4], L[:, 4:])
    lo = jnp.minimum(L[:, :4], L[:, 4:])
    L = jnp.concatenate([hi, lo], axis=1)
    parts = []
    for s in (0, 4):
        x0 = L[:, s:s + 2]
        x1 = L[:, s + 2:s + 4]
        parts += [jnp.maximum(x0, x1), jnp.minimum(x0, x1)]
    L = jnp.concatenate(parts, axis=1)
    parts = []
    for s in (0, 2, 4, 6):
        x0 = L[:, s:s + 1]
        x1 = L[:, s + 1:s + 2]
        parts += [jnp.maximum(x0, x1), jnp.minimum(x0, x1)]
    return jnp.concatenate(parts, axis=1)


def kernel(x):
    rows, n = x.shape

    def body(x_ref, out_ref, comm_ref, send_sem, recv_sem):
        my_x = lax.axis_index("x")
        my_y = lax.axis_index("y")
        nbr = (1 - my_x, my_y)

        barrier_sem = pltpu.get_barrier_semaphore()
        pl.semaphore_signal(
            barrier_sem, inc=1, device_id=nbr,
            device_id_type=pl.DeviceIdType.MESH,
        )

        local = _topk_desc_distinct(_distinctify(x_ref[:, :]), K)
        comm_ref[0, :, :] = local

        pl.semaphore_wait(barrier_sem, 1)
        rdma = pltpu.make_async_remote_copy(
            src_ref=comm_ref.at[0],
            dst_ref=comm_ref.at[1],
            send_sem=send_sem,
            recv_sem=recv_sem,
            device_id=nbr,
            device_id_type=pl.DeviceIdType.MESH,
        )
        rdma.start()
        rdma.wait()

        out_ref[:, :] = _merge_sorted8(local, comm_ref[1, :, :])

    return pl.pallas_call(
        body,
        out_shape=jax.ShapeDtypeStruct((rows, K), jnp.float32),
        in_specs=[pl.BlockSpec(memory_space=pltpu.VMEM)],
        out_specs=pl.BlockSpec(memory_space=pltpu.VMEM),
        scratch_shapes=[
            pltpu.VMEM((2, rows, K), jnp.float32),
            pltpu.SemaphoreType.DMA,
            pltpu.SemaphoreType.DMA,
        ],
        compiler_params=pltpu.CompilerParams(collective_id=0),
    )(x)
